# baseline (speedup 1.0000x reference)
import functools

import jax
import jax.numpy as jnp
from jax import lax
from jax.experimental import pallas as pl
from jax.experimental.pallas import tpu as pltpu
from jax.experimental.pallas import tpu_sc as plsc

_SMOOTHING = 0.1
_FLAT_C = 2048
_BLK_R = 2000


def _gather_kernel(pred_rows, tgt_hbm, out_hbm, tgt_v, idx_v, rows_v, part_v,
                   zero_v, sem, *, n_classes, b_per_w, num_cores):
    wid = lax.axis_index("s") * num_cores + lax.axis_index("c")
    base = wid * b_per_w
    n_chunks = b_per_w // 16

    pltpu.sync_copy(tgt_hbm.at[pl.ds(base, b_per_w)], tgt_v)

    for k in range(n_chunks):
        t = tgt_v[pl.ds(k * 16, 16)]
        rowg = (base + k * 16) + lax.iota(jnp.int32, 16)
        flat = rowg * n_classes + t
        idx_v[pl.ds(k * 16, 16)] = lax.shift_right_logical(flat, 7)

    pltpu.async_copy(pred_rows.at[idx_v], rows_v, sem).wait()

    acc = jnp.zeros((16,), jnp.float32)
    lane_iota = lax.iota(jnp.int32, 16)
    for k in range(n_chunks):
        t_vec = tgt_v[pl.ds(k * 16, 16)]
        rowg = (base + k * 16) + lax.iota(jnp.int32, 16)
        lane_vec = lax.bitwise_and(rowg * n_classes + t_vec, 127)
        sub_vec = lax.bitwise_and(lane_vec, 15)
        blk_vec = lax.shift_right_logical(lane_vec, 4)
        for m in range(16):
            j = k * 16 + m
            val16 = rows_v[j, pl.ds(blk_vec[m] * 16, 16)]
            acc = acc + jnp.where(lane_iota == sub_vec[m], val16, 0.0)

    part_v[...] = acc
    zero_v[...] = jnp.zeros((16,), jnp.float32)
    pltpu.sync_copy(part_v, out_hbm.at[pl.ds(wid * 16, 16)])
    pltpu.sync_copy(zero_v, out_hbm.at[pl.ds(512 + wid * 16, 16)])


def _sum_kernel(pred_ref, scpart_ref, out_ref, acc_ref, *, n_steps, eps,
                conf_m_eps, inv_b):
    j = pl.program_id(0)

    @pl.when(j == 0)
    def _():
        acc_ref[0] = jnp.float32(0.0)

    acc_ref[0] += jnp.sum(pred_ref[...])

    @pl.when(j == n_steps - 1)
    def _():
        g = jnp.sum(scpart_ref[...])
        out_ref[0, 0] = -(eps * acc_ref[0] + conf_m_eps * g) * inv_b


@jax.jit
def kernel(pred, target):
    b, c = pred.shape
    eps = _SMOOTHING / (c - 1)
    conf = 1.0 - _SMOOTHING

    info = plsc.get_sparse_core_info()
    num_workers = info.num_cores * info.num_subcores
    b_per_w = b // num_workers

    pred_rows = pred.reshape(b * c // 128, 128)
    mesh = plsc.VectorSubcoreMesh(core_axis_name="c", subcore_axis_name="s")
    gathered = pl.kernel(
        functools.partial(
            _gather_kernel,
            n_classes=c,
            b_per_w=b_per_w,
            num_cores=info.num_cores,
        ),
        mesh=mesh,
        out_type=jax.ShapeDtypeStruct((1024,), jnp.float32),
        scratch_types=[
            pltpu.VMEM((b_per_w,), jnp.int32),
            pltpu.VMEM((b_per_w,), jnp.int32),
            pltpu.VMEM((b_per_w, 128), jnp.float32),
            pltpu.VMEM((16,), jnp.float32),
            pltpu.VMEM((16,), jnp.float32),
            pltpu.SemaphoreType.DMA,
        ],
    )(pred_rows, target.astype(jnp.int32))

    flat = pred.reshape(b * c // _FLAT_C, _FLAT_C)
    n_steps = flat.shape[0] // _BLK_R
    scpart = gathered.reshape(8, 128)

    loss = pl.pallas_call(
        functools.partial(
            _sum_kernel,
            n_steps=n_steps,
            eps=eps,
            conf_m_eps=conf - eps,
            inv_b=1.0 / b,
        ),
        grid=(n_steps,),
        in_specs=[
            pl.BlockSpec((_BLK_R, _FLAT_C), lambda j: (j, 0)),
            pl.BlockSpec((8, 128), lambda j: (0, 0)),
        ],
        out_specs=pl.BlockSpec(
            (1, 1), lambda j: (0, 0), memory_space=pltpu.SMEM
        ),
        out_shape=jax.ShapeDtypeStruct((1, 1), jnp.float32),
        scratch_shapes=[pltpu.SMEM((1,), jnp.float32)],
        compiler_params=pltpu.CompilerParams(
            dimension_semantics=("arbitrary",),
        ),
    )(flat, scpart)

    return loss[0, 0]


# speedup vs baseline: 3.8101x; 3.8101x over previous
import functools

import jax
import jax.numpy as jnp
from jax import lax
from jax.experimental import pallas as pl
from jax.experimental.pallas import tpu as pltpu
from jax.experimental.pallas import tpu_sc as plsc

_SMOOTHING = 0.1
_BLK_C = 2048


def _gather_kernel(pred_hbm, tgt_hbm, out_hbm, tgt_v, rows_v, part_v,
                   zero_v, sem, *, b_per_w, num_cores):
    wid = lax.axis_index("s") * num_cores + lax.axis_index("c")
    base = wid * b_per_w
    n_chunks = b_per_w // 16

    pltpu.sync_copy(tgt_hbm.at[pl.ds(base, b_per_w)], tgt_v)

    copies = []
    for k in range(n_chunks):
        t_vec = tgt_v[pl.ds(k * 16, 16)]
        col0_vec = lax.bitwise_and(t_vec, ~127)
        for m in range(16):
            j = k * 16 + m
            copies.append(
                pltpu.async_copy(
                    pred_hbm.at[
                        pl.ds(base + 8 * (j // 8), 8),
                        pl.ds(pl.multiple_of(col0_vec[m], 128), 128),
                    ],
                    rows_v.at[j],
                    sem,
                )
            )
    for cp in copies:
        cp.wait()

    acc = jnp.zeros((16,), jnp.float32)
    lane_iota = lax.iota(jnp.int32, 16)
    for k in range(n_chunks):
        t_vec = tgt_v[pl.ds(k * 16, 16)]
        lane_vec = lax.bitwise_and(t_vec, 127)
        sub_vec = lax.bitwise_and(lane_vec, 15)
        blk_vec = lax.shift_right_logical(lane_vec, 4)
        for m in range(16):
            j = k * 16 + m
            val16 = rows_v[j, j % 8, pl.ds(blk_vec[m] * 16, 16)]
            acc = acc + jnp.where(lane_iota == sub_vec[m], val16, 0.0)

    part_v[...] = acc
    zero_v[...] = jnp.zeros((16,), jnp.float32)
    pltpu.sync_copy(part_v, out_hbm.at[pl.ds(wid * 16, 16)])
    pltpu.sync_copy(zero_v, out_hbm.at[pl.ds(512 + wid * 16, 16)])


def _sum_kernel(pred_ref, scpart_ref, out_ref, acc_ref, *, n_classes, blk_c,
                n_steps, eps, conf_m_eps, inv_b):
    j = pl.program_id(0)

    @pl.when(j == 0)
    def _():
        acc_ref[0] = jnp.float32(0.0)

    x = pred_ref[...]
    cols = j * blk_c + lax.broadcasted_iota(jnp.int32, x.shape, 1)
    acc_ref[0] += jnp.sum(jnp.where(cols < n_classes, x, jnp.float32(0.0)))

    @pl.when(j == n_steps - 1)
    def _():
        g = jnp.sum(scpart_ref[...])
        out_ref[0, 0] = -(eps * acc_ref[0] + conf_m_eps * g) * inv_b


@jax.jit
def kernel(pred, target):
    b, c = pred.shape
    eps = _SMOOTHING / (c - 1)
    conf = 1.0 - _SMOOTHING

    info = plsc.get_sparse_core_info()
    num_workers = info.num_cores * info.num_subcores
    b_per_w = b // num_workers

    mesh = plsc.VectorSubcoreMesh(core_axis_name="c", subcore_axis_name="s")
    gathered = pl.kernel(
        functools.partial(
            _gather_kernel,
            b_per_w=b_per_w,
            num_cores=info.num_cores,
        ),
        mesh=mesh,
        out_type=jax.ShapeDtypeStruct((1024,), jnp.float32),
        scratch_types=[
            pltpu.VMEM((b_per_w,), jnp.int32),
            pltpu.VMEM((b_per_w, 8, 128), jnp.float32),
            pltpu.VMEM((16,), jnp.float32),
            pltpu.VMEM((16,), jnp.float32),
            pltpu.SemaphoreType.DMA,
        ],
    )(pred, target.astype(jnp.int32))

    n_steps = pl.cdiv(c, _BLK_C)
    scpart = gathered.reshape(8, 128)

    loss = pl.pallas_call(
        functools.partial(
            _sum_kernel,
            n_classes=c,
            blk_c=_BLK_C,
            n_steps=n_steps,
            eps=eps,
            conf_m_eps=conf - eps,
            inv_b=1.0 / b,
        ),
        grid=(n_steps,),
        in_specs=[
            pl.BlockSpec((b, _BLK_C), lambda j: (0, j)),
            pl.BlockSpec((8, 128), lambda j: (0, 0)),
        ],
        out_specs=pl.BlockSpec(
            (1, 1), lambda j: (0, 0), memory_space=pltpu.SMEM
        ),
        out_shape=jax.ShapeDtypeStruct((1, 1), jnp.float32),
        scratch_shapes=[pltpu.SMEM((1,), jnp.float32)],
        compiler_params=pltpu.CompilerParams(
            dimension_semantics=("arbitrary",),
        ),
    )(pred, scpart)

    return loss[0, 0]


# speedup vs baseline: 13.3053x; 3.4921x over previous
import functools

import jax
import jax.numpy as jnp
from jax import lax
from jax.experimental import pallas as pl
from jax.experimental.pallas import tpu as pltpu
from jax.experimental.pallas import tpu_sc as plsc

_SMOOTHING = 0.1
_BLK_R = 6144
_SC_ROWS = 49152
_CHUNK_R = 32


def _sc_kernel(pt_hbm, tgt_hbm, out_hbm, tgt_v, rows_v, part_v, buf0, buf1,
               gsem, sem0, sem1, *, n_batch, b_per_w, num_cores, sc_rows,
               chunk_r):
    wid = lax.axis_index("s") * num_cores + lax.axis_index("c")
    base = wid * b_per_w
    n_tgt_chunks = b_per_w // 16

    pltpu.sync_copy(tgt_hbm.at[pl.ds(base, b_per_w)], tgt_v)

    col0 = pl.multiple_of((base // 128) * 128, 128)
    gather_copies = []
    for k in range(n_tgt_chunks):
        t_vec = tgt_v[pl.ds(k * 16, 16)]
        row0_vec = lax.bitwise_and(t_vec, ~7)
        for m in range(16):
            j = k * 16 + m
            gather_copies.append(
                pltpu.async_copy(
                    pt_hbm.at[
                        pl.ds(pl.multiple_of(row0_vec[m], 8), 8),
                        pl.ds(col0, 128),
                    ],
                    rows_v.at[j],
                    gsem,
                )
            )

    rows_per_w = sc_rows // 32
    n_chunks = rows_per_w // chunk_r
    dense_base = wid * rows_per_w
    bufs = (buf0, buf1)
    sems = (sem0, sem1)
    pltpu.async_copy(
        pt_hbm.at[pl.ds(dense_base, chunk_r), pl.ds(0, n_batch)], buf0, sem0
    )
    pltpu.async_copy(
        pt_hbm.at[pl.ds(dense_base + chunk_r, chunk_r), pl.ds(0, n_batch)],
        buf1, sem1,
    )

    n_acc = 4
    n_sub = n_batch // 16

    def outer_body(g, accs):
        for parity in range(2):
            c = 2 * g + parity
            buf, sem = bufs[parity], sems[parity]
            pltpu.make_async_copy(
                pt_hbm.at[pl.ds(dense_base, chunk_r), pl.ds(0, n_batch)],
                buf, sem,
            ).wait()

            def row_body(r, a):
                a = list(a)
                for l in range(n_sub):
                    a[l % n_acc] = a[l % n_acc] + buf[r, pl.ds(l * 16, 16)]
                return tuple(a)

            accs = lax.fori_loop(0, chunk_r, row_body, accs)

            @pl.when(c + 2 < n_chunks)
            def _():
                off = pl.multiple_of(
                    dense_base + (c + 2) * chunk_r, chunk_r
                )
                pltpu.async_copy(
                    pt_hbm.at[pl.ds(off, chunk_r), pl.ds(0, n_batch)],
                    buf, sem,
                )
        return accs

    accs = tuple(jnp.zeros((16,), jnp.float32) for _ in range(n_acc))
    accs = lax.fori_loop(0, n_chunks // 2, outer_body, accs)
    dense_acc = accs[0] + accs[1] + accs[2] + accs[3]

    for cp in gather_copies:
        cp.wait()

    gacc = jnp.zeros((16,), jnp.float32)
    lane_iota = lax.iota(jnp.int32, 16)
    blk0 = (base % 128) // 16
    for k in range(n_tgt_chunks):
        t_vec = tgt_v[pl.ds(k * 16, 16)]
        rsub_vec = lax.bitwise_and(t_vec, 7)
        for m in range(16):
            j = k * 16 + m
            bs = blk0 + j // 16
            code = (j % 16) + 16 * rsub_vec[m]
            for r in range(8):
                val16 = rows_v[j, r, pl.ds(bs * 16, 16)]
                hit = (lane_iota + 16 * r) == code
                gacc = gacc + jnp.where(hit, val16, 0.0)

    part_v[...] = gacc
    pltpu.sync_copy(part_v, out_hbm.at[pl.ds(wid * 16, 16)])
    part_v[...] = dense_acc
    pltpu.sync_copy(part_v, out_hbm.at[pl.ds(512 + wid * 16, 16)])


def _sum_kernel(pt_ref, out_ref, acc_ref, *, n_rows_total, row0, blk_r,
                n_steps):
    j = pl.program_id(0)

    @pl.when(j == 0)
    def _():
        acc_ref[0] = jnp.float32(0.0)

    x = pt_ref[...]
    rows = row0 + j * blk_r + lax.broadcasted_iota(jnp.int32, x.shape, 0)
    acc_ref[0] += jnp.sum(jnp.where(rows < n_rows_total, x, jnp.float32(0.0)))

    @pl.when(j == n_steps - 1)
    def _():
        out_ref[0, 0] = acc_ref[0]


def _combine_kernel(scpart_ref, tcsum_ref, out_ref, *, eps, conf_m_eps,
                    inv_b):
    p = scpart_ref[...]
    g = jnp.sum(p[0:4, :])
    s_sc = jnp.sum(p[4:8, :])
    out_ref[0, 0] = -(eps * (tcsum_ref[0, 0] + s_sc) + conf_m_eps * g) * inv_b


@jax.jit
def kernel(pred, target):
    b, c = pred.shape
    eps = _SMOOTHING / (c - 1)
    conf = 1.0 - _SMOOTHING

    pt = pred.T

    info = plsc.get_sparse_core_info()
    num_workers = info.num_cores * info.num_subcores
    b_per_w = b // num_workers

    mesh = plsc.VectorSubcoreMesh(core_axis_name="c", subcore_axis_name="s")
    scpart = pl.kernel(
        functools.partial(
            _sc_kernel,
            n_batch=b,
            b_per_w=b_per_w,
            num_cores=info.num_cores,
            sc_rows=_SC_ROWS,
            chunk_r=_CHUNK_R,
        ),
        mesh=mesh,
        out_type=jax.ShapeDtypeStruct((1024,), jnp.float32),
        scratch_types=[
            pltpu.VMEM((b_per_w,), jnp.int32),
            pltpu.VMEM((b_per_w, 8, 128), jnp.float32),
            pltpu.VMEM((16,), jnp.float32),
            pltpu.VMEM((_CHUNK_R, 1024), jnp.float32),
            pltpu.VMEM((_CHUNK_R, 1024), jnp.float32),
            pltpu.SemaphoreType.DMA,
            pltpu.SemaphoreType.DMA,
            pltpu.SemaphoreType.DMA,
        ],
    )(pt, target.astype(jnp.int32))

    tc_rows = c - _SC_ROWS
    n_steps = pl.cdiv(tc_rows, _BLK_R)
    row_blk0 = _SC_ROWS // _BLK_R

    tcsum = pl.pallas_call(
        functools.partial(
            _sum_kernel,
            n_rows_total=c,
            row0=_SC_ROWS,
            blk_r=_BLK_R,
            n_steps=n_steps,
        ),
        grid=(n_steps,),
        in_specs=[
            pl.BlockSpec((_BLK_R, b), lambda j: (j + row_blk0, 0)),
        ],
        out_specs=pl.BlockSpec(
            (1, 1), lambda j: (0, 0), memory_space=pltpu.SMEM
        ),
        out_shape=jax.ShapeDtypeStruct((1, 1), jnp.float32),
        scratch_shapes=[pltpu.SMEM((1,), jnp.float32)],
        compiler_params=pltpu.CompilerParams(
            dimension_semantics=("arbitrary",),
        ),
    )(pt)

    loss = pl.pallas_call(
        functools.partial(
            _combine_kernel,
            eps=eps,
            conf_m_eps=conf - eps,
            inv_b=1.0 / b,
        ),
        in_specs=[
            pl.BlockSpec((8, 128), lambda: (0, 0)),
            pl.BlockSpec((1, 1), lambda: (0, 0), memory_space=pltpu.SMEM),
        ],
        out_specs=pl.BlockSpec(
            (1, 1), lambda: (0, 0), memory_space=pltpu.SMEM
        ),
        out_shape=jax.ShapeDtypeStruct((1, 1), jnp.float32),
    )(scpart.reshape(8, 128), tcsum)

    return loss[0, 0]
